# Initial kernel scaffold; baseline (speedup 1.0000x reference)
import functools

import jax
import jax.numpy as jnp
from jax import lax
from jax.experimental import pallas as pl
from jax.experimental.pallas import tpu as pltpu
from jax.experimental.pallas import tpu_sc as plsc

N = 10000
E = 320000
D = 128
CH = 125
NCH = E // CH
NT = 16
NC = 2
RPT = N // NT
ZR = 125
BN = 2000


def _sc_agg_build(do_counts, edge_split, table_rows):
    cpt = NCH // (NC * NT) if edge_split else NCH // NT
    mesh = plsc.VectorSubcoreMesh(core_axis_name="c", subcore_axis_name="s")

    out_type = [jax.ShapeDtypeStruct((NC * N, D), jnp.float32)]
    scratch = [
        pltpu.VMEM((cpt, CH), jnp.int32),
        pltpu.VMEM((cpt, CH), jnp.int32),
        pltpu.VMEM((CH, D), jnp.float32),
        pltpu.VMEM_SHARED((N, D), jnp.float32),
        pltpu.SemaphoreType.DMA,
    ]
    if do_counts:
        out_type.append(jax.ShapeDtypeStruct((NC * N, 16), jnp.float32))
        scratch += [
            pltpu.VMEM((CH, 16), jnp.float32),
            pltpu.VMEM_SHARED((N, 16), jnp.float32),
        ]

    def body(table, srcx, dstx, zrows, zcnt, ones, out, *rest):
        if do_counts:
            cnt_out, srcv, dstv, rowbuf, agg_s, sem, onesv, cnt_s = rest
        else:
            srcv, dstv, rowbuf, agg_s, sem = rest
        c = lax.axis_index("c")
        s = lax.axis_index("s")

        if edge_split:
            row0 = (c * NT + s) * cpt
        else:
            row0 = s * cpt
        pltpu.sync_copy(srcx.at[c, pl.ds(row0, cpt)], srcv)
        pltpu.sync_copy(dstx.at[c, pl.ds(row0, cpt)], dstv)

        for z in range(RPT // ZR):
            pltpu.sync_copy(zrows, agg_s.at[pl.ds(s * RPT + z * ZR, ZR)])
        if do_counts:
            pltpu.sync_copy(zcnt, cnt_s.at[pl.ds(s * RPT, RPT)])
            pltpu.sync_copy(ones, onesv)
        plsc.subcore_barrier()

        def step(j, carry):
            pltpu.async_copy(table.at[srcv.at[j]], rowbuf, sem).wait()
            pltpu.sync_copy(rowbuf, agg_s.at[dstv.at[j]], add=True)
            if do_counts:
                pltpu.sync_copy(onesv, cnt_s.at[dstv.at[j]], add=True)
            return carry

        lax.fori_loop(0, cpt, step, 0)
        plsc.subcore_barrier()

        pltpu.sync_copy(agg_s.at[pl.ds(s * RPT, RPT)],
                        out.at[pl.ds(c * N + s * RPT, RPT)])
        if do_counts:
            pltpu.sync_copy(cnt_s.at[pl.ds(s * RPT, RPT)],
                            cnt_out.at[pl.ds(c * N + s * RPT, RPT)])

    return pl.kernel(body, out_type=tuple(out_type), mesh=mesh,
                     scratch_types=tuple(scratch))


def _dense1_body(aggp, cntp, x, wl, wr, b, h_out):
    cnt = cntp[0, :, 0] + cntp[1, :, 0]
    inv = 1.0 / jnp.maximum(cnt, 1.0)
    mean = (aggp[0] + aggp[1]) * inv[:, None]
    h = (jnp.dot(mean, wl[...], preferred_element_type=jnp.float32)
         + jnp.dot(x[...], wr[...], preferred_element_type=jnp.float32)
         + b[0][None, :])
    h = jnp.maximum(h, 0.0)
    h_out[0] = h[:, :D]
    h_out[1] = h[:, D:]


def _dense2_body(agg2, cntp, h, wlm, wrm, bm, wll, wrl, bl, mu_out, ls_out):
    cnt = cntp[0, :, 0] + cntp[1, :, 0]
    inv = 1.0 / jnp.maximum(cnt, 1.0)
    mean2 = jnp.concatenate([agg2[0], agg2[1]], axis=1) * inv[:, None]
    hcat = jnp.concatenate([h[0], h[1]], axis=1)
    mu_out[...] = (jnp.dot(mean2, wlm[...], preferred_element_type=jnp.float32)
                   + jnp.dot(hcat, wrm[...], preferred_element_type=jnp.float32)
                   + bm[0][None, :])
    ls_out[...] = (jnp.dot(mean2, wll[...], preferred_element_type=jnp.float32)
                   + jnp.dot(hcat, wrl[...], preferred_element_type=jnp.float32)
                   + bl[0][None, :])


def _full(shape):
    return pl.BlockSpec(shape, lambda i: tuple(0 for _ in shape))


def _rows3(shape):
    return pl.BlockSpec(shape, lambda i: (0, i, 0))


_sc_agg_l1 = _sc_agg_build(do_counts=True, edge_split=True, table_rows=N)
_sc_agg_l2 = _sc_agg_build(do_counts=False, edge_split=False, table_rows=2 * N)

_dense1 = pl.pallas_call(
    _dense1_body,
    grid=(N // BN,),
    in_specs=[
        _rows3((2, BN, D)),
        _rows3((2, BN, 16)),
        pl.BlockSpec((BN, D), lambda i: (i, 0)),
        _full((D, 2 * D)), _full((D, 2 * D)), _full((1, 2 * D)),
    ],
    out_specs=_rows3((2, BN, D)),
    out_shape=jax.ShapeDtypeStruct((2, N, D), jnp.float32),
)

_dense2 = pl.pallas_call(
    _dense2_body,
    grid=(N // BN,),
    in_specs=[
        _rows3((2, BN, D)),
        _rows3((2, BN, 16)),
        _rows3((2, BN, D)),
        _full((2 * D, D)), _full((2 * D, D)), _full((1, D)),
        _full((2 * D, D)), _full((2 * D, D)), _full((1, D)),
    ],
    out_specs=[pl.BlockSpec((BN, D), lambda i: (i, 0)),
               pl.BlockSpec((BN, D), lambda i: (i, 0))],
    out_shape=[jax.ShapeDtypeStruct((N, D), jnp.float32),
               jax.ShapeDtypeStruct((N, D), jnp.float32)],
)


@jax.jit
def kernel(x, edge_index, W_l1, W_r1, b1, W_lmu, W_rmu, b_mu, W_lls, W_rls, b_ls):
    src = edge_index[0].astype(jnp.int32).reshape(NCH, CH)
    dst = edge_index[1].astype(jnp.int32).reshape(NCH, CH)
    srcx1 = jnp.stack([src, src])
    srcx2 = jnp.stack([src, src + N])
    dstx = jnp.stack([dst, dst])
    zrows = jnp.zeros((ZR, D), jnp.float32)
    zcnt = jnp.zeros((RPT, 16), jnp.float32)
    ones = jnp.ones((CH, 16), jnp.float32)

    agg1p, cntp = _sc_agg_l1(x, srcx1, dstx, zrows, zcnt, ones)
    cntp = cntp.reshape(2, N, 16)
    h = _dense1(agg1p.reshape(2, N, D), cntp, x, W_l1, W_r1,
                b1.reshape(1, -1))
    agg2 = _sc_agg_l2(h.reshape(2 * N, D), srcx2, dstx, zrows, zcnt, ones)
    mu, ls = _dense2(agg2.reshape(2, N, D), cntp, h,
                     W_lmu, W_rmu, b_mu.reshape(1, -1),
                     W_lls, W_rls, b_ls.reshape(1, -1))
    return (mu, ls)


# speedup vs baseline: 2.6695x; 2.6695x over previous
import functools

import jax
import jax.numpy as jnp
from jax import lax
from jax.experimental import pallas as pl
from jax.experimental.pallas import tpu as pltpu
from jax.experimental.pallas import tpu_sc as plsc

N = 10000
E = 320000
D = 128
CH = 128
NCH = 2560
EPAD = NCH * CH - E
NT = 16
NC = 2
NS = N + 8
RPT = 624
TAIL = N - NT * RPT
BN = 2000

_MESH = plsc.VectorSubcoreMesh(core_axis_name="c", subcore_axis_name="s")


def _zero_fill(rowbuf, s):
    zv = jnp.zeros((16,), jnp.float32)

    def fillz(i, carry):
        for u in range(D // 16):
            rowbuf[i, pl.ds(u * 16, 16)] = zv
        return carry
    lax.fori_loop(0, CH, fillz, 0)

    def do_zero(agg_s):
        for z in range(RPT // CH):
            pltpu.sync_copy(rowbuf, agg_s.at[pl.ds(s * RPT + z * CH, CH)])
        rem = RPT - (RPT // CH) * CH
        pltpu.sync_copy(rowbuf.at[pl.ds(0, rem)],
                        agg_s.at[pl.ds(s * RPT + (RPT // CH) * CH, rem)])

        @pl.when(s == NT - 1)
        def _zero_tail():
            pltpu.sync_copy(rowbuf.at[pl.ds(0, TAIL + 8)],
                            agg_s.at[pl.ds(NT * RPT, TAIL + 8)])

    return do_zero


def _writeback(agg_s, out, c, s):
    pltpu.sync_copy(agg_s.at[pl.ds(s * RPT, RPT)],
                    out.at[pl.ds(c * N + s * RPT, RPT)])

    @pl.when(s == NT - 1)
    def _copy_tail():
        pltpu.sync_copy(agg_s.at[pl.ds(NT * RPT, TAIL)],
                        out.at[pl.ds(c * N + NT * RPT, TAIL)])


def _sc_cnt_build():
    cpt = NCH // (NC * NT)
    scratch = (
        pltpu.VMEM((CH,), jnp.int32),
        pltpu.VMEM((CH, D), jnp.float32),
        pltpu.VMEM_SHARED((NS, D), jnp.float32),
    )

    def body(dstx, out, dstv, rowbuf, cnt_s):
        c = lax.axis_index("c")
        s = lax.axis_index("s")
        row0 = (c * NT + s) * cpt

        _zero_fill(rowbuf, s)(cnt_s)

        ov = jnp.ones((16,), jnp.float32)

        def fillo(i, carry):
            for u in range(D // 16):
                rowbuf[i, pl.ds(u * 16, 16)] = ov
            return carry
        lax.fori_loop(0, CH, fillo, 0)

        plsc.subcore_barrier()

        def step(j, carry):
            pltpu.sync_copy(dstx.at[c, row0 + j], dstv)
            pltpu.sync_copy(rowbuf, cnt_s.at[dstv], add=True)
            return carry
        lax.fori_loop(0, cpt, step, 0)

        plsc.subcore_barrier()
        _writeback(cnt_s, out, c, s)

    return pl.kernel(body,
                     out_type=(jax.ShapeDtypeStruct((NC * N, D), jnp.float32),),
                     mesh=_MESH, scratch_types=scratch)


def _sc_agg_build(edge_split):
    cpt = NCH // (NC * NT) if edge_split else NCH // NT
    scratch = (
        pltpu.VMEM((CH,), jnp.int32),
        pltpu.VMEM((CH,), jnp.int32),
        pltpu.VMEM((CH, D), jnp.float32),
        pltpu.VMEM_SHARED((NS, D), jnp.float32),
        pltpu.SemaphoreType.DMA,
    )

    def body(table, srcx, dstx, out, srcv, dstv, rowbuf, agg_s, sem):
        c = lax.axis_index("c")
        s = lax.axis_index("s")
        row0 = (c * NT + s) * cpt if edge_split else s * cpt

        _zero_fill(rowbuf, s)(agg_s)
        plsc.subcore_barrier()

        def step(j, carry):
            pltpu.sync_copy(srcx.at[c, row0 + j], srcv)
            pltpu.sync_copy(dstx.at[c, row0 + j], dstv)
            pltpu.async_copy(table.at[srcv], rowbuf, sem).wait()
            pltpu.sync_copy(rowbuf, agg_s.at[dstv], add=True)
            return carry
        lax.fori_loop(0, cpt, step, 0)

        plsc.subcore_barrier()
        _writeback(agg_s, out, c, s)

    return pl.kernel(body,
                     out_type=(jax.ShapeDtypeStruct((NC * N, D), jnp.float32),),
                     mesh=_MESH, scratch_types=scratch)


def _dense1_body(cntp, m1p, x, wl, wr, b, h_out):
    cnt = cntp[0, :, 0] + cntp[1, :, 0]
    inv = 1.0 / jnp.maximum(cnt, 1.0)
    mean = (m1p[0] + m1p[1]) * inv[:, None]
    h = (jnp.dot(mean, wl[...], preferred_element_type=jnp.float32)
         + jnp.dot(x[...], wr[...], preferred_element_type=jnp.float32)
         + b[0][None, :])
    h = jnp.maximum(h, 0.0)
    h_out[0] = h[:, :D]
    h_out[1] = h[:, D:]


def _dense2_body(cntp, m2, h, wlm, wrm, bm, wll, wrl, bl, mu_out, ls_out):
    cnt = cntp[0, :, 0] + cntp[1, :, 0]
    inv = 1.0 / jnp.maximum(cnt, 1.0)
    mean2 = jnp.concatenate([m2[0], m2[1]], axis=1) * inv[:, None]
    hcat = jnp.concatenate([h[0], h[1]], axis=1)
    mu_out[...] = (jnp.dot(mean2, wlm[...], preferred_element_type=jnp.float32)
                   + jnp.dot(hcat, wrm[...], preferred_element_type=jnp.float32)
                   + bm[0][None, :])
    ls_out[...] = (jnp.dot(mean2, wll[...], preferred_element_type=jnp.float32)
                   + jnp.dot(hcat, wrl[...], preferred_element_type=jnp.float32)
                   + bl[0][None, :])


def _full(shape):
    return pl.BlockSpec(shape, lambda i: tuple(0 for _ in shape))


def _rows3(shape):
    return pl.BlockSpec(shape, lambda i: (0, i, 0))


_sc_cnt = _sc_cnt_build()
_sc_agg_l1 = _sc_agg_build(edge_split=True)
_sc_agg_l2 = _sc_agg_build(edge_split=False)

_dense1 = pl.pallas_call(
    _dense1_body,
    grid=(N // BN,),
    in_specs=[
        _rows3((2, BN, D)),
        _rows3((2, BN, D)),
        pl.BlockSpec((BN, D), lambda i: (i, 0)),
        _full((D, 2 * D)), _full((D, 2 * D)), _full((1, 2 * D)),
    ],
    out_specs=_rows3((2, BN, D)),
    out_shape=jax.ShapeDtypeStruct((2, N, D), jnp.float32),
)

_dense2 = pl.pallas_call(
    _dense2_body,
    grid=(N // BN,),
    in_specs=[
        _rows3((2, BN, D)),
        _rows3((2, BN, D)),
        _rows3((2, BN, D)),
        _full((2 * D, D)), _full((2 * D, D)), _full((1, D)),
        _full((2 * D, D)), _full((2 * D, D)), _full((1, D)),
    ],
    out_specs=[pl.BlockSpec((BN, D), lambda i: (i, 0)),
               pl.BlockSpec((BN, D), lambda i: (i, 0))],
    out_shape=[jax.ShapeDtypeStruct((N, D), jnp.float32),
               jax.ShapeDtypeStruct((N, D), jnp.float32)],
)


@jax.jit
def kernel(x, edge_index, W_l1, W_r1, b1, W_lmu, W_rmu, b_mu, W_lls, W_rls, b_ls):
    src = edge_index[0].astype(jnp.int32)
    dst = edge_index[1].astype(jnp.int32)
    srcp = jnp.concatenate([src, jnp.zeros((EPAD,), jnp.int32)]).reshape(NCH, CH)
    dstp = jnp.concatenate([dst, jnp.full((EPAD,), N, jnp.int32)]).reshape(NCH, CH)
    srcx1 = jnp.stack([srcp, srcp])
    srcx2 = jnp.stack([srcp, srcp + N])
    dstx = jnp.stack([dstp, dstp])

    cntp, = _sc_cnt(dstx)
    m1p, = _sc_agg_l1(x, srcx1, dstx)
    cntp = cntp.reshape(2, N, D)
    h = _dense1(cntp, m1p.reshape(2, N, D), x, W_l1, W_r1, b1.reshape(1, -1))
    m2, = _sc_agg_l2(h.reshape(2 * N, D), srcx2, dstx)
    mu, ls = _dense2(cntp, m2.reshape(2, N, D), h,
                     W_lmu, W_rmu, b_mu.reshape(1, -1),
                     W_lls, W_rls, b_ls.reshape(1, -1))
    return (mu, ls)


# speedup vs baseline: 3.1628x; 1.1848x over previous
import functools

import jax
import jax.numpy as jnp
from jax import lax
from jax.experimental import pallas as pl
from jax.experimental.pallas import tpu as pltpu
from jax.experimental.pallas import tpu_sc as plsc

N = 10000
E = 320000
D = 128
CH = 128
NCH = 2560
EPAD = NCH * CH - E
NT = 16
NC = 2
NS = N + 8
RPT = 624
TAIL = N - NT * RPT
BN = 2000

_MESH = plsc.VectorSubcoreMesh(core_axis_name="c", subcore_axis_name="s")


def _zero_fill(rowbuf, s):
    zv = jnp.zeros((16,), jnp.float32)

    def fillz(i, carry):
        for u in range(D // 16):
            rowbuf[i, pl.ds(u * 16, 16)] = zv
        return carry
    lax.fori_loop(0, CH, fillz, 0)

    def do_zero(agg_s):
        for z in range(RPT // CH):
            pltpu.sync_copy(rowbuf, agg_s.at[pl.ds(s * RPT + z * CH, CH)])
        rem = RPT - (RPT // CH) * CH
        pltpu.sync_copy(rowbuf.at[pl.ds(0, rem)],
                        agg_s.at[pl.ds(s * RPT + (RPT // CH) * CH, rem)])

        @pl.when(s == NT - 1)
        def _zero_tail():
            pltpu.sync_copy(rowbuf.at[pl.ds(0, TAIL + 8)],
                            agg_s.at[pl.ds(NT * RPT, TAIL + 8)])

    return do_zero


def _writeback(agg_s, out, c, s):
    pltpu.sync_copy(agg_s.at[pl.ds(s * RPT, RPT)],
                    out.at[pl.ds(c * N + s * RPT, RPT)])

    @pl.when(s == NT - 1)
    def _copy_tail():
        pltpu.sync_copy(agg_s.at[pl.ds(NT * RPT, TAIL)],
                        out.at[pl.ds(c * N + NT * RPT, TAIL)])


G = 8


def _sc_cnt_build():
    cpt = NCH // (NC * NT)
    scratch = (
        pltpu.VMEM((G, CH), jnp.int32),
        pltpu.VMEM((CH, D), jnp.float32),
        pltpu.VMEM_SHARED((NS, D), jnp.float32),
        pltpu.SemaphoreType.DMA,
    )

    def body(dstx, out, didx, rowbuf, cnt_s, ssem):
        c = lax.axis_index("c")
        s = lax.axis_index("s")
        row0 = (c * NT + s) * cpt

        _zero_fill(rowbuf, s)(cnt_s)

        ov = jnp.ones((16,), jnp.float32)

        def fillo(i, carry):
            for u in range(D // 16):
                rowbuf[i, pl.ds(u * 16, 16)] = ov
            return carry
        lax.fori_loop(0, CH, fillo, 0)

        plsc.subcore_barrier()

        def group(g, carry):
            pltpu.sync_copy(dstx.at[c, pl.ds(row0 + g * G, G)], didx)
            descs = [pltpu.async_copy(rowbuf, cnt_s.at[didx.at[b]], ssem,
                                      add=True)
                     for b in range(G)]
            for d in descs:
                d.wait()
            return carry
        lax.fori_loop(0, cpt // G, group, 0)

        plsc.subcore_barrier()
        _writeback(cnt_s, out, c, s)

    return pl.kernel(body,
                     out_type=(jax.ShapeDtypeStruct((NC * N, D), jnp.float32),),
                     mesh=_MESH, scratch_types=scratch)


def _sc_agg_build(edge_split):
    cpt = NCH // (NC * NT) if edge_split else NCH // NT
    scratch = (
        pltpu.VMEM((G, CH), jnp.int32),
        pltpu.VMEM((G, CH), jnp.int32),
        pltpu.VMEM((CH, D), jnp.float32),
        pltpu.VMEM((CH, D), jnp.float32),
        pltpu.VMEM_SHARED((NS, D), jnp.float32),
        pltpu.SemaphoreType.DMA,
        pltpu.SemaphoreType.DMA,
        pltpu.SemaphoreType.DMA,
        pltpu.SemaphoreType.DMA,
    )

    def body(table, srcx, dstx, out, sidx, didx, rba, rbb, agg_s,
             gsa, gsb, ssa, ssb):
        c = lax.axis_index("c")
        s = lax.axis_index("s")
        row0 = (c * NT + s) * cpt if edge_split else s * cpt

        _zero_fill(rba, s)(agg_s)
        plsc.subcore_barrier()

        def group(g, carry):
            base = row0 + g * G
            pltpu.sync_copy(srcx.at[c, pl.ds(base, G)], sidx)
            pltpu.sync_copy(dstx.at[c, pl.ds(base, G)], didx)
            for h in range(G // 2):
                b0, b1 = 2 * h, 2 * h + 1
                g0 = pltpu.async_copy(table.at[sidx.at[b0]], rba, gsa)
                g1 = pltpu.async_copy(table.at[sidx.at[b1]], rbb, gsb)
                g0.wait()
                s0 = pltpu.async_copy(rba, agg_s.at[didx.at[b0]], ssa,
                                      add=True)
                g1.wait()
                s1 = pltpu.async_copy(rbb, agg_s.at[didx.at[b1]], ssb,
                                      add=True)
                s0.wait()
                s1.wait()
            return carry
        lax.fori_loop(0, cpt // G, group, 0)

        plsc.subcore_barrier()
        _writeback(agg_s, out, c, s)

    return pl.kernel(body,
                     out_type=(jax.ShapeDtypeStruct((NC * N, D), jnp.float32),),
                     mesh=_MESH, scratch_types=scratch)


def _dense1_body(cntp, m1p, x, wl, wr, b, h_out):
    cnt = cntp[0, :, 0] + cntp[1, :, 0]
    inv = 1.0 / jnp.maximum(cnt, 1.0)
    mean = (m1p[0] + m1p[1]) * inv[:, None]
    h = (jnp.dot(mean, wl[...], preferred_element_type=jnp.float32)
         + jnp.dot(x[...], wr[...], preferred_element_type=jnp.float32)
         + b[0][None, :])
    h = jnp.maximum(h, 0.0)
    h_out[0] = h[:, :D]
    h_out[1] = h[:, D:]


def _dense2_body(cntp, m2, h, wlm, wrm, bm, wll, wrl, bl, mu_out, ls_out):
    cnt = cntp[0, :, 0] + cntp[1, :, 0]
    inv = 1.0 / jnp.maximum(cnt, 1.0)
    mean2 = jnp.concatenate([m2[0], m2[1]], axis=1) * inv[:, None]
    hcat = jnp.concatenate([h[0], h[1]], axis=1)
    mu_out[...] = (jnp.dot(mean2, wlm[...], preferred_element_type=jnp.float32)
                   + jnp.dot(hcat, wrm[...], preferred_element_type=jnp.float32)
                   + bm[0][None, :])
    ls_out[...] = (jnp.dot(mean2, wll[...], preferred_element_type=jnp.float32)
                   + jnp.dot(hcat, wrl[...], preferred_element_type=jnp.float32)
                   + bl[0][None, :])


def _full(shape):
    return pl.BlockSpec(shape, lambda i: tuple(0 for _ in shape))


def _rows3(shape):
    return pl.BlockSpec(shape, lambda i: (0, i, 0))


_sc_cnt = _sc_cnt_build()
_sc_agg_l1 = _sc_agg_build(edge_split=True)
_sc_agg_l2 = _sc_agg_build(edge_split=False)

_dense1 = pl.pallas_call(
    _dense1_body,
    grid=(N // BN,),
    in_specs=[
        _rows3((2, BN, D)),
        _rows3((2, BN, D)),
        pl.BlockSpec((BN, D), lambda i: (i, 0)),
        _full((D, 2 * D)), _full((D, 2 * D)), _full((1, 2 * D)),
    ],
    out_specs=_rows3((2, BN, D)),
    out_shape=jax.ShapeDtypeStruct((2, N, D), jnp.float32),
)

_dense2 = pl.pallas_call(
    _dense2_body,
    grid=(N // BN,),
    in_specs=[
        _rows3((2, BN, D)),
        _rows3((2, BN, D)),
        _rows3((2, BN, D)),
        _full((2 * D, D)), _full((2 * D, D)), _full((1, D)),
        _full((2 * D, D)), _full((2 * D, D)), _full((1, D)),
    ],
    out_specs=[pl.BlockSpec((BN, D), lambda i: (i, 0)),
               pl.BlockSpec((BN, D), lambda i: (i, 0))],
    out_shape=[jax.ShapeDtypeStruct((N, D), jnp.float32),
               jax.ShapeDtypeStruct((N, D), jnp.float32)],
)


@jax.jit
def kernel(x, edge_index, W_l1, W_r1, b1, W_lmu, W_rmu, b_mu, W_lls, W_rls, b_ls):
    src = edge_index[0].astype(jnp.int32)
    dst = edge_index[1].astype(jnp.int32)
    srcp = jnp.concatenate([src, jnp.zeros((EPAD,), jnp.int32)]).reshape(NCH, CH)
    dstp = jnp.concatenate([dst, jnp.full((EPAD,), N, jnp.int32)]).reshape(NCH, CH)
    srcx1 = jnp.stack([srcp, srcp])
    srcx2 = jnp.stack([srcp, srcp + N])
    dstx = jnp.stack([dstp, dstp])

    cntp, = _sc_cnt(dstx)
    m1p, = _sc_agg_l1(x, srcx1, dstx)
    cntp = cntp.reshape(2, N, D)
    h = _dense1(cntp, m1p.reshape(2, N, D), x, W_l1, W_r1, b1.reshape(1, -1))
    m2, = _sc_agg_l2(h.reshape(2 * N, D), srcx2, dstx)
    mu, ls = _dense2(cntp, m2.reshape(2, N, D), h,
                     W_lmu, W_rmu, b_mu.reshape(1, -1),
                     W_lls, W_rls, b_ls.reshape(1, -1))
    return (mu, ls)
